# Initial kernel scaffold; baseline (speedup 1.0000x reference)
import jax
import jax.numpy as jnp
from jax.experimental import pallas as pl


def kernel(input_ids, tok_table, pos_table):
    raise NotImplementedError("write your pallas kernel here")


# speedup vs baseline: 1.3113x; 1.3113x over previous
import functools

import jax
import jax.numpy as jnp
from jax import lax
from jax.experimental import pallas as pl
from jax.experimental.pallas import tpu as pltpu
from jax.experimental.pallas import tpu_sc as plsc

_B, _S, _D = 4, 4096, 1024
_N = _B * _S
_NW = 32
_RPW = _N // _NW
_C = 32
_NCH = _RPW // _C
_LANES = 16


def _embed_body(ids_hbm, tok_hbm, pos_hbm, out_hbm,
                idx_v, tok_buf, pos_buf, sem_t, sem_p):
    cid = lax.axis_index("c")
    sid = lax.axis_index("s")
    wid = sid * 2 + cid
    base = wid * _RPW
    pos_base = lax.rem(base, _S)

    pltpu.sync_copy(ids_hbm.at[pl.ds(base, _RPW)], idx_v)

    def chunk(ci, carry):
        off = ci * _C
        cp_t = pltpu.make_async_copy(
            tok_hbm.at[idx_v.at[pl.ds(off, _C)]], tok_buf, sem_t)
        cp_p = pltpu.make_async_copy(
            pos_hbm.at[pl.ds(pos_base + off, _C)], pos_buf, sem_p)
        cp_t.start()
        cp_p.start()
        cp_t.wait()
        cp_p.wait()

        def row(r, c2):
            for k in range(_D // _LANES):
                sl = pl.ds(k * _LANES, _LANES)
                tok_buf[r, sl] = tok_buf[r, sl] + pos_buf[r, sl]
            return c2
        lax.fori_loop(0, _C, row, 0)

        pltpu.sync_copy(tok_buf, out_hbm.at[pl.ds(base + off, _C)])
        return carry

    lax.fori_loop(0, _NCH, chunk, 0)


_embed_kernel = functools.partial(
    pl.kernel,
    out_type=jax.ShapeDtypeStruct((_N, _D), jnp.float32),
    mesh=plsc.VectorSubcoreMesh(core_axis_name="c", subcore_axis_name="s"),
    scratch_types=[
        pltpu.VMEM((_RPW,), jnp.int32),
        pltpu.VMEM((_C, _D), jnp.float32),
        pltpu.VMEM((_C, _D), jnp.float32),
        pltpu.SemaphoreType.DMA,
        pltpu.SemaphoreType.DMA,
    ],
)(_embed_body)


def kernel(input_ids, tok_table, pos_table):
    ids = input_ids.reshape(-1).astype(jnp.int32)
    out = _embed_kernel(ids, tok_table, pos_table)
    return out.reshape(_B, _S, _D)


# speedup vs baseline: 2.0979x; 1.5999x over previous
import functools

import jax
import jax.numpy as jnp
from jax import lax
from jax.experimental import pallas as pl
from jax.experimental.pallas import tpu as pltpu
from jax.experimental.pallas import tpu_sc as plsc

_B, _S, _D = 4, 4096, 1024
_N = _B * _S
_NW = 32
_SPW = _S // _NW
_C = 16
_NSC = _SPW // _C
_STEPS = _NSC * _B
_LANES = 16


def _embed_body(ids_hbm, tok_hbm, pos_hbm, out_hbm,
                idx_v, tb0, tb1, pb0, pb1,
                gsem0, gsem1, psem0, psem1, ssem0, ssem1):
    cid = lax.axis_index("c")
    sid = lax.axis_index("s")
    wid = sid * 2 + cid
    s_base = wid * _SPW

    tb = (tb0, tb1)
    pb = (pb0, pb1)
    gsem = (gsem0, gsem1)
    psem = (psem0, psem1)
    ssem = (ssem0, ssem1)

    for b in range(_B):
        pltpu.sync_copy(ids_hbm.at[pl.ds(b * _S + s_base, _SPW)],
                        idx_v.at[pl.ds(b * _SPW, _SPW)])

    def start_gather(t):
        sc, b = t // _B, t % _B
        idx = idx_v.at[pl.ds(b * _SPW + sc * _C, _C)]
        return pltpu.async_copy(tok_hbm.at[idx], tb[t % 2], gsem[t % 2])

    def start_pos(sc):
        return pltpu.async_copy(pos_hbm.at[pl.ds(s_base + sc * _C, _C)],
                                pb[sc % 2], psem[sc % 2])

    g_desc = [None] * _STEPS
    s_desc = [None] * _STEPS
    p_desc = [None] * _NSC
    p_desc[0] = start_pos(0)
    g_desc[0] = start_gather(0)

    for t in range(_STEPS):
        sc, b = t // _B, t % _B
        if t + 1 < _STEPS:
            if t >= 1:
                s_desc[t - 1].wait()
            g_desc[t + 1] = start_gather(t + 1)
        if b == 0:
            if sc + 1 < _NSC:
                p_desc[sc + 1] = start_pos(sc + 1)
            p_desc[sc].wait()
        g_desc[t].wait()

        tbuf, pbuf = tb[t % 2], pb[sc % 2]

        def add_half(i, carry, tbuf=tbuf, pbuf=pbuf):
            r = i // 2
            h = (i % 2) * (_D // 2)
            for k in range(_D // (2 * _LANES)):
                sl = pl.ds(h + k * _LANES, _LANES)
                tbuf[r, sl] = tbuf[r, sl] + pbuf[r, sl]
            return carry
        lax.fori_loop(0, 2 * _C, add_half, 0)

        s_desc[t] = pltpu.async_copy(
            tbuf, out_hbm.at[pl.ds(b * _S + s_base + sc * _C, _C)],
            ssem[t % 2])

    s_desc[_STEPS - 2].wait()
    s_desc[_STEPS - 1].wait()


_embed_kernel = functools.partial(
    pl.kernel,
    out_type=jax.ShapeDtypeStruct((_N, _D), jnp.float32),
    mesh=plsc.VectorSubcoreMesh(core_axis_name="c", subcore_axis_name="s"),
    scratch_types=[
        pltpu.VMEM((_B * _SPW,), jnp.int32),
        pltpu.VMEM((_C, _D), jnp.float32),
        pltpu.VMEM((_C, _D), jnp.float32),
        pltpu.VMEM((_C, _D), jnp.float32),
        pltpu.VMEM((_C, _D), jnp.float32),
        pltpu.SemaphoreType.DMA,
        pltpu.SemaphoreType.DMA,
        pltpu.SemaphoreType.DMA,
        pltpu.SemaphoreType.DMA,
        pltpu.SemaphoreType.DMA,
        pltpu.SemaphoreType.DMA,
    ],
)(_embed_body)


def kernel(input_ids, tok_table, pos_table):
    ids = input_ids.reshape(-1).astype(jnp.int32)
    out = _embed_kernel(ids, tok_table, pos_table)
    return out.reshape(_B, _S, _D)


# speedup vs baseline: 2.3664x; 1.1280x over previous
import functools

import jax
import jax.numpy as jnp
from jax import lax
from jax.experimental import pallas as pl
from jax.experimental.pallas import tpu as pltpu
from jax.experimental.pallas import tpu_sc as plsc

_B, _S, _D = 4, 4096, 1024
_N = _B * _S
_NW = 32
_SPW = _S // _NW
_C = 16
_NSC = _SPW // _C
_STEPS = _NSC * _B
_NBUF = 4
_LANES = 16


def _embed_body(ids_hbm, tok_hbm, pos_hbm, out_hbm,
                idx_v, tbufs, pbufs, gsems, psems, ssems):
    cid = lax.axis_index("c")
    sid = lax.axis_index("s")
    wid = sid * 2 + cid
    s_base = wid * _SPW

    for b in range(_B):
        pltpu.sync_copy(ids_hbm.at[pl.ds(b * _S + s_base, _SPW)],
                        idx_v.at[pl.ds(b * _SPW, _SPW)])

    def start_gather(t):
        sc, b = t // _B, t % _B
        idx = idx_v.at[pl.ds(b * _SPW + sc * _C, _C)]
        return pltpu.async_copy(tok_hbm.at[idx], tbufs[t % _NBUF],
                                gsems[t % _NBUF])

    def start_pos(sc):
        return pltpu.async_copy(pos_hbm.at[pl.ds(s_base + sc * _C, _C)],
                                pbufs[sc % 2], psems[sc % 2])

    g_desc = [None] * _STEPS
    s_desc = [None] * _STEPS
    p_desc = [None] * _NSC
    p_desc[0] = start_pos(0)
    for t in range(_NBUF - 1):
        g_desc[t] = start_gather(t)

    for t in range(_STEPS):
        sc, b = t // _B, t % _B
        if b == 0:
            if sc + 1 < _NSC:
                p_desc[sc + 1] = start_pos(sc + 1)
            p_desc[sc].wait()
        g_desc[t].wait()

        tbuf, pbuf = tbufs[t % _NBUF], pbufs[sc % 2]

        def add_half(i, carry, tbuf=tbuf, pbuf=pbuf):
            r = i // 2
            h = (i % 2) * (_D // 2)
            for k in range(_D // (2 * _LANES)):
                sl = pl.ds(h + k * _LANES, 16)
                tbuf[r, sl] = tbuf[r, sl] + pbuf[r, sl]
            return carry
        lax.fori_loop(0, 2 * _C, add_half, 0)

        s_desc[t] = pltpu.async_copy(
            tbuf, out_hbm.at[pl.ds(b * _S + s_base + sc * _C, _C)],
            ssems[t % _NBUF])

        if t + _NBUF - 1 < _STEPS:
            if t >= 1:
                s_desc[t - 1].wait()
            g_desc[t + _NBUF - 1] = start_gather(t + _NBUF - 1)

    for t in range(_STEPS - _NBUF + 1, _STEPS):
        s_desc[t].wait()


_embed_kernel = functools.partial(
    pl.kernel,
    out_type=jax.ShapeDtypeStruct((_N, _D), jnp.float32),
    mesh=plsc.VectorSubcoreMesh(core_axis_name="c", subcore_axis_name="s"),
    scratch_types=[
        pltpu.VMEM((_B * _SPW,), jnp.int32),
        tuple(pltpu.VMEM((_C, _D), jnp.float32) for _ in range(_NBUF)),
        tuple(pltpu.VMEM((_C, _D), jnp.float32) for _ in range(2)),
        tuple(pltpu.SemaphoreType.DMA for _ in range(_NBUF)),
        tuple(pltpu.SemaphoreType.DMA for _ in range(2)),
        tuple(pltpu.SemaphoreType.DMA for _ in range(_NBUF)),
    ],
)(_embed_body)


def kernel(input_ids, tok_table, pos_table):
    ids = input_ids.reshape(-1).astype(jnp.int32)
    out = _embed_kernel(ids, tok_table, pos_table)
    return out.reshape(_B, _S, _D)
